# baseline (speedup 1.0000x reference)
import functools

import jax
import jax.numpy as jnp
from jax import lax
from jax.experimental import pallas as pl
from jax.experimental.pallas import tpu as pltpu
from jax.experimental.pallas import tpu_sc as plsc

_NC = 2
_NS = 16
_NW = _NC * _NS

_V = 1000000
_D = 32
_B = 16384
_F = 26
_N = _B * _F


_C1 = 4096


def _t1_body(x_ref, o_ref):
    y = x_ref[...].T
    y = y.reshape(_C1 // 4, 4, 32)
    o_ref[...] = jnp.concatenate(
        [y[:, 0, :], y[:, 1, :], y[:, 2, :], y[:, 3, :]], axis=1)


def _table_relayout(tblT):
    return pl.pallas_call(
        _t1_body,
        grid=((_V + _C1 - 1) // _C1,),
        in_specs=[pl.BlockSpec((32, _C1), lambda i: (0, i))],
        out_specs=pl.BlockSpec((_C1 // 4, 128), lambda i: (i, 0)),
        out_shape=jax.ShapeDtypeStruct((_V // 4, 128), jnp.float32),
    )(tblT)


@functools.lru_cache(maxsize=None)
def _make_gather(N, D, CH, NBUF):
    n_per_w = N // _NW
    n_chunks = n_per_w // CH
    assert n_chunks * CH == n_per_w, (N, CH)
    mesh = plsc.VectorSubcoreMesh(core_axis_name="c", subcore_axis_name="s")

    @functools.partial(
        pl.kernel,
        mesh=mesh,
        out_type=jax.ShapeDtypeStruct((N, D), jnp.float32),
        scratch_types=[
            pltpu.VMEM((n_per_w,), jnp.int32),
            *[pltpu.VMEM((CH, D), jnp.float32) for _ in range(NBUF)],
            *[pltpu.SemaphoreType.DMA for _ in range(2 * NBUF)],
        ],
        compiler_params=pltpu.CompilerParams(use_tc_tiling_on_sc=False),
    )
    def k(idx_hbm, tbl_hbm, out_hbm, idx_all, *bufs_and_sems):
        rows = bufs_and_sems[:NBUF]
        sem_g = bufs_and_sems[NBUF:2 * NBUF]
        sem_w = bufs_and_sems[2 * NBUF:]
        wid = lax.axis_index("s") * _NC + lax.axis_index("c")
        base = pl.multiple_of(wid * n_per_w, 8)
        pltpu.sync_copy(idx_hbm.at[pl.ds(base, n_per_w)], idx_all)

        def start_gather(i):
            return pltpu.async_copy(
                tbl_hbm.at[idx_all.at[pl.ds(i * CH, CH)]],
                rows[i % NBUF], sem_g[i % NBUF])

        gathers = [None] * n_chunks
        wbs = [None] * n_chunks
        for i in range(min(NBUF - 1, n_chunks)):
            gathers[i] = start_gather(i)
        for i in range(n_chunks):
            b = i % NBUF
            gathers[i].wait()
            off = pl.multiple_of(base + i * CH, 8)
            wbs[i] = pltpu.async_copy(
                rows[b], out_hbm.at[pl.ds(off, CH)], sem_w[b])
            nxt = i + NBUF - 1
            if nxt < n_chunks:
                if i >= 1:
                    wbs[i - 1].wait()
                gathers[nxt] = start_gather(nxt)
        for i in range(max(0, n_chunks - NBUF), n_chunks):
            if wbs[i] is not None:
                wbs[i].wait()

    return k


_Q = 512


def _t3_body(x_ref, o_ref):
    x = x_ref[...]
    outs = [x[:, 32 * a:32 * a + 32].T for a in range(4)]
    z = jnp.stack(outs, axis=2)
    o_ref[...] = z.reshape(1, 32, _Q * 4)


def _out_relayout(g2):
    blocks_per_field = (_B // 4) // _Q
    return pl.pallas_call(
        _t3_body,
        grid=(_F * blocks_per_field,),
        in_specs=[pl.BlockSpec((_Q, 128), lambda g: (g, 0))],
        out_specs=pl.BlockSpec(
            (1, 32, _Q * 4),
            lambda g: (g // blocks_per_field, 0, g % blocks_per_field)),
        out_shape=jax.ShapeDtypeStruct((_F, _D, _B), jnp.float32),
    )(g2)


def kernel(inputs, embedding):
    tblT = embedding.T
    idx = inputs.T.reshape(_N)
    tbl_rows = _table_relayout(tblT).reshape(_V, _D)
    flat = _make_gather(_N, _D, 832, 4)(idx, tbl_rows)
    g2 = flat.reshape(_N * _D // 128, 128)
    out = _out_relayout(g2)
    return out.transpose(2, 0, 1)


# speedup vs baseline: 4.4426x; 4.4426x over previous
import functools

import jax
import jax.numpy as jnp
from jax import lax
from jax.experimental import pallas as pl
from jax.experimental.pallas import tpu as pltpu
from jax.experimental.pallas import tpu_sc as plsc

_NC = 2
_NS = 16
_NW = _NC * _NS

_V = 1000000
_D = 32
_B = 16384
_F = 26
_N = _B * _F


_C1 = 4096


def _t1_body(x_ref, o_ref):
    y = x_ref[...].T
    y = y.reshape(_C1 // 4, 4, 32)
    o_ref[...] = jnp.concatenate(
        [y[:, 0, :], y[:, 1, :], y[:, 2, :], y[:, 3, :]], axis=1)


def _table_relayout(tblT):
    return pl.pallas_call(
        _t1_body,
        grid=((_V + _C1 - 1) // _C1,),
        in_specs=[pl.BlockSpec((32, _C1), lambda i: (0, i))],
        out_specs=pl.BlockSpec((_C1 // 4, 128), lambda i: (i, 0)),
        out_shape=jax.ShapeDtypeStruct((_V // 4, 128), jnp.float32),
    )(tblT)


@functools.lru_cache(maxsize=None)
def _make_gather(N, D, CH, NBUF):
    n_per_w = N // _NW
    n_chunks = n_per_w // CH
    assert n_chunks * CH == n_per_w, (N, CH)
    mesh = plsc.VectorSubcoreMesh(core_axis_name="c", subcore_axis_name="s")

    @functools.partial(
        pl.kernel,
        mesh=mesh,
        out_type=jax.ShapeDtypeStruct((N, D), jnp.float32),
        scratch_types=[
            pltpu.VMEM((n_per_w,), jnp.int32),
            *[pltpu.VMEM((CH, D), jnp.float32) for _ in range(NBUF)],
            *[pltpu.SemaphoreType.DMA for _ in range(2 * NBUF)],
        ],
        compiler_params=pltpu.CompilerParams(use_tc_tiling_on_sc=False),
    )
    def k(idx_hbm, tbl_hbm, out_hbm, idx_all, *bufs_and_sems):
        rows = bufs_and_sems[:NBUF]
        sem_g = bufs_and_sems[NBUF:2 * NBUF]
        sem_w = bufs_and_sems[2 * NBUF:]
        wid = lax.axis_index("s") * _NC + lax.axis_index("c")
        base = pl.multiple_of(wid * n_per_w, 8)
        pltpu.sync_copy(idx_hbm.at[pl.ds(base, n_per_w)], idx_all)

        def start_gather(i):
            return pltpu.async_copy(
                tbl_hbm.at[idx_all.at[pl.ds(i * CH, CH)]],
                rows[i % NBUF], sem_g[i % NBUF])

        gathers = [None] * n_chunks
        wbs = [None] * n_chunks
        for i in range(min(NBUF - 1, n_chunks)):
            gathers[i] = start_gather(i)
        for i in range(n_chunks):
            b = i % NBUF
            gathers[i].wait()
            off = pl.multiple_of(base + i * CH, 8)
            wbs[i] = pltpu.async_copy(
                rows[b], out_hbm.at[pl.ds(off, CH)], sem_w[b])
            nxt = i + NBUF - 1
            if nxt < n_chunks:
                if i >= 1:
                    wbs[i - 1].wait()
                gathers[nxt] = start_gather(nxt)
        for i in range(max(0, n_chunks - NBUF), n_chunks):
            if wbs[i] is not None:
                wbs[i].wait()

    return k


def kernel(inputs, embedding):
    tblT = embedding.T
    idx = inputs.reshape(_N)
    tbl_rows = _table_relayout(tblT).reshape(_V, _D)
    flat = _make_gather(_N, _D, 832, 4)(idx, tbl_rows)
    return flat.reshape(_B, _F, _D)


# speedup vs baseline: 4.4978x; 1.0124x over previous
import functools

import jax
import jax.numpy as jnp
from jax import lax
from jax.experimental import pallas as pl
from jax.experimental.pallas import tpu as pltpu
from jax.experimental.pallas import tpu_sc as plsc

_NC = 2
_NS = 16
_NW = _NC * _NS

_V = 1000000
_D = 32
_B = 16384
_F = 26
_N = _B * _F


_C1 = 8192


def _t1_body(x_ref, o_ref):
    y = x_ref[...].T
    y = y.reshape(_C1 // 4, 4, 32)
    o_ref[...] = jnp.concatenate(
        [y[:, 0, :], y[:, 1, :], y[:, 2, :], y[:, 3, :]], axis=1)


def _table_relayout(tblT):
    return pl.pallas_call(
        _t1_body,
        grid=((_V + _C1 - 1) // _C1,),
        in_specs=[pl.BlockSpec((32, _C1), lambda i: (0, i))],
        out_specs=pl.BlockSpec((_C1 // 4, 128), lambda i: (i, 0)),
        out_shape=jax.ShapeDtypeStruct((_V // 4, 128), jnp.float32),
    )(tblT)


@functools.lru_cache(maxsize=None)
def _make_gather(N, D, CH, NBUF):
    n_per_w = N // _NW
    n_chunks = n_per_w // CH
    assert n_chunks * CH == n_per_w, (N, CH)
    mesh = plsc.VectorSubcoreMesh(core_axis_name="c", subcore_axis_name="s")

    @functools.partial(
        pl.kernel,
        mesh=mesh,
        out_type=jax.ShapeDtypeStruct((N, D), jnp.float32),
        scratch_types=[
            pltpu.VMEM((n_per_w,), jnp.int32),
            *[pltpu.VMEM((CH, D), jnp.float32) for _ in range(NBUF)],
            *[pltpu.SemaphoreType.DMA for _ in range(2 * NBUF)],
        ],
        compiler_params=pltpu.CompilerParams(use_tc_tiling_on_sc=False),
    )
    def k(idx_hbm, tbl_hbm, out_hbm, idx_all, *bufs_and_sems):
        rows = bufs_and_sems[:NBUF]
        sem_g = bufs_and_sems[NBUF:2 * NBUF]
        sem_w = bufs_and_sems[2 * NBUF:]
        wid = lax.axis_index("s") * _NC + lax.axis_index("c")
        base = pl.multiple_of(wid * n_per_w, 8)
        pltpu.sync_copy(idx_hbm.at[pl.ds(base, n_per_w)], idx_all)

        def start_gather(i):
            return pltpu.async_copy(
                tbl_hbm.at[idx_all.at[pl.ds(i * CH, CH)]],
                rows[i % NBUF], sem_g[i % NBUF])

        gathers = [None] * n_chunks
        wbs = [None] * n_chunks
        for i in range(min(NBUF - 1, n_chunks)):
            gathers[i] = start_gather(i)
        for i in range(n_chunks):
            b = i % NBUF
            gathers[i].wait()
            off = pl.multiple_of(base + i * CH, 8)
            wbs[i] = pltpu.async_copy(
                rows[b], out_hbm.at[pl.ds(off, CH)], sem_w[b])
            nxt = i + NBUF - 1
            if nxt < n_chunks:
                if i >= 1:
                    wbs[i - 1].wait()
                gathers[nxt] = start_gather(nxt)
        for i in range(max(0, n_chunks - NBUF), n_chunks):
            if wbs[i] is not None:
                wbs[i].wait()

    return k


def kernel(inputs, embedding):
    tblT = embedding.T
    idx = inputs.reshape(_N)
    tbl_rows = _table_relayout(tblT).reshape(_V, _D)
    flat = _make_gather(_N, _D, 832, 4)(idx, tbl_rows)
    return flat.reshape(_B, _F, _D)
